# baseline (speedup 1.0000x reference)
import functools

import jax
import jax.numpy as jnp
from jax import lax
from jax.experimental import pallas as pl
from jax.experimental.pallas import tpu as pltpu

_BN_EPS = 1e-5


def _pick_bs(hw, max_bs):
    bs = max_bs
    while bs > 8 and hw % bs != 0:
        bs //= 2
    return bs if hw % bs == 0 else hw


def _stats_kernel(x_ref, prm_ref, fold_ref, acc_ref, *, c, bs, p):
    i = pl.program_id(0)

    @pl.when(i == 0)
    def _():
        acc_ref[...] = jnp.zeros_like(acc_ref)

    xs = [x_ref[j] for j in range(c)]
    planes = xs + [xs[a] * xs[b]
                   for a in range(c) for b in range(a, c)]
    for m, v in enumerate(planes):
        r = v[0:8]
        for j in range(8, bs, 8):
            r = r + v[j:j + 8]
        acc_ref[8 * m:8 * m + 8] += r

    @pl.when(i == pl.num_programs(0) - 1)
    def _():
        inv_p = 1.0 / p
        sums = [jnp.sum(acc_ref[8 * m:8 * m + 8, :]) for m in range(
            c + c * (c + 1) // 2)]
        mu = [sums[a] * inv_p for a in range(c)]
        cov = {}
        idx = c
        for a in range(c):
            for b in range(a, c):
                cov[(a, b)] = cov[(b, a)] = sums[idx] * inv_p - mu[a] * mu[b]
                idx += 1
        prm = prm_ref[...]
        w1c = [prm[:, a:a + 1] for a in range(c)]
        b1 = prm[:, c:c + 1]
        gamma = prm[:, c + 1:c + 2]
        beta = prm[:, c + 2:c + 3]
        mean_y = b1
        for a in range(c):
            mean_y = mean_y + w1c[a] * mu[a]
        var_y = jnp.zeros_like(b1)
        for a in range(c):
            for b in range(c):
                var_y = var_y + (w1c[a] * w1c[b]) * cov[(a, b)]
        scale = gamma * lax.rsqrt(var_y + _BN_EPS)
        for a in range(c):
            fold_ref[:, a:a + 1] = w1c[a] * scale
        fold_ref[:, c:c + 1] = scale * (b1 - mean_y) + beta


def _main_kernel(x_ref, w1a_ref, w2b_ref, aux_ref, o_ref, *, c, bs):
    xv = x_ref[...].reshape(c * bs, x_ref.shape[2])
    ones = jnp.ones((1, xv.shape[1]), jnp.float32)
    xa = jnp.concatenate([xv, ones], axis=0).astype(jnp.bfloat16)
    h = jnp.dot(w1a_ref[...], xa,
                preferred_element_type=jnp.float32)
    r = jnp.maximum(h.astype(jnp.bfloat16), 0)
    y = jnp.dot(w2b_ref[...], r,
                preferred_element_type=jnp.float32)
    o_ref[...] = jnp.tanh(y + aux_ref[0:1, 0:1])


@jax.jit
def _guide_nn_opt(x_nchw, w1, b1, gamma, beta, w2, b2):
    n, c, hh, ww = x_nchw.shape
    k = w1.shape[0]
    hw = hh * ww
    p = n * hw

    xp = jnp.transpose(x_nchw.astype(jnp.float32),
                       (1, 2, 3, 0)).reshape(c, hw, n)

    bs1 = _pick_bs(hw, 128)
    nm = c + c * (c + 1) // 2
    prm = jnp.concatenate(
        [w1, b1, gamma, beta,
         jnp.zeros((k, 8 - c - 3), jnp.float32)], axis=1)
    fold, _ = pl.pallas_call(
        functools.partial(_stats_kernel, c=c, bs=bs1, p=float(p)),
        out_shape=(jax.ShapeDtypeStruct((k, 128), jnp.float32),
                   jax.ShapeDtypeStruct((8 * nm, n), jnp.float32)),
        grid=(hw // bs1,),
        in_specs=[pl.BlockSpec((c, bs1, n), lambda i: (0, i, 0)),
                  pl.BlockSpec((k, 8), lambda i: (0, 0))],
        out_specs=(pl.BlockSpec((k, 128), lambda i: (0, 0)),
                   pl.BlockSpec((8 * nm, n), lambda i: (0, 0))),
        compiler_params=pltpu.CompilerParams(
            dimension_semantics=("arbitrary",)),
    )(xp, prm)

    w1f = fold[:, 0:c]
    b1f = fold[:, c:c + 1]

    bs2 = _pick_bs(hw, 64)
    eye = jnp.eye(bs2, dtype=jnp.float32)
    w1a = jnp.concatenate(
        [jnp.kron(w1f, eye), jnp.repeat(b1f, bs2, axis=0)],
        axis=1).astype(jnp.bfloat16)
    w2b = jnp.kron(w2.T, eye).astype(jnp.bfloat16)
    aux = jnp.broadcast_to(b2.astype(jnp.float32), (8, 128))

    outp = pl.pallas_call(
        functools.partial(_main_kernel, c=c, bs=bs2),
        out_shape=jax.ShapeDtypeStruct((hw, n), jnp.float32),
        grid=(hw // bs2,),
        in_specs=[
            pl.BlockSpec((c, bs2, n), lambda i: (0, i, 0)),
            pl.BlockSpec((k * bs2, c * bs2 + 1), lambda i: (0, 0)),
            pl.BlockSpec((bs2, k * bs2), lambda i: (0, 0)),
            pl.BlockSpec((8, 128), lambda i: (0, 0)),
        ],
        out_specs=pl.BlockSpec((bs2, n), lambda i: (i, 0)),
        compiler_params=pltpu.CompilerParams(
            dimension_semantics=("parallel",)),
    )(xp, w1a, w2b, aux)

    return outp.reshape(1, hh, ww, n).transpose(3, 0, 1, 2)


def kernel(x_nchw, w1, b1, gamma, beta, w2, b2):
    return _guide_nn_opt(x_nchw, w1, b1, gamma, beta, w2, b2)


# speedup vs baseline: 1.0929x; 1.0929x over previous
import functools

import jax
import jax.numpy as jnp
from jax import lax
from jax.experimental import pallas as pl
from jax.experimental.pallas import tpu as pltpu

_BN_EPS = 1e-5


def _pick_bs(hw, max_bs):
    bs = max_bs
    while bs > 8 and hw % bs != 0:
        bs //= 2
    return bs if hw % bs == 0 else hw


def _stats_kernel(x_ref, acc_ref, *, c, bs):
    i = pl.program_id(0)

    @pl.when(i == 0)
    def _():
        acc_ref[...] = jnp.zeros_like(acc_ref)

    xs = [x_ref[j] for j in range(c)]
    planes = xs + [xs[a] * xs[b]
                   for a in range(c) for b in range(a, c)]
    for m, v in enumerate(planes):
        r = v[0:8]
        for j in range(8, bs, 8):
            r = r + v[j:j + 8]
        acc_ref[8 * m:8 * m + 8] += r


def _main_kernel(x_ref, w1a_ref, w2b_ref, aux_ref, o_ref, *, c, bs):
    xv = x_ref[...].reshape(c * bs, x_ref.shape[2])
    ones = jnp.ones((1, xv.shape[1]), jnp.float32)
    xa = jnp.concatenate([xv, ones], axis=0).astype(jnp.bfloat16)
    h = jnp.dot(w1a_ref[...], xa,
                preferred_element_type=jnp.float32)
    r = jnp.maximum(h.astype(jnp.bfloat16), 0)
    y = jnp.dot(w2b_ref[...], r,
                preferred_element_type=jnp.float32)
    o_ref[...] = jnp.tanh(y + aux_ref[0:1, 0:1])


@jax.jit
def _guide_nn_opt(x_nchw, w1, b1, gamma, beta, w2, b2):
    n, c, hh, ww = x_nchw.shape
    k = w1.shape[0]
    hw = hh * ww
    p = n * hw

    xp = jnp.transpose(x_nchw.astype(jnp.float32),
                       (1, 2, 3, 0)).reshape(c, hw, n)

    bs1 = _pick_bs(hw, 256)
    nm = c + c * (c + 1) // 2
    acc = pl.pallas_call(
        functools.partial(_stats_kernel, c=c, bs=bs1),
        out_shape=jax.ShapeDtypeStruct((8 * nm, n), jnp.float32),
        grid=(hw // bs1,),
        in_specs=[pl.BlockSpec((c, bs1, n), lambda i: (0, i, 0))],
        out_specs=pl.BlockSpec((8 * nm, n), lambda i: (0, 0)),
        compiler_params=pltpu.CompilerParams(
            dimension_semantics=("arbitrary",)),
    )(xp)

    gv = acc.reshape(nm, 8 * n).sum(axis=1)
    s = gv[:c]
    pairs = {}
    idx = c
    for a in range(c):
        for b in range(a, c):
            pairs[(a, b)] = pairs[(b, a)] = gv[idx]
            idx += 1
    q = jnp.stack([jnp.stack([pairs[(a, b)] for b in range(c)])
                   for a in range(c)])

    mu = s / p
    cov = q / p - mu[:, None] * mu[None, :]
    mean_y = w1 @ mu[:, None] + b1
    var_y = jnp.sum((w1 @ cov) * w1, axis=1, keepdims=True)

    scale = gamma * lax.rsqrt(var_y + _BN_EPS)
    w1f = w1 * scale
    b1f = scale * (b1 - mean_y) + beta

    bs2 = _pick_bs(hw, 64)
    eye = jnp.eye(bs2, dtype=jnp.float32)
    w1a = jnp.concatenate(
        [jnp.kron(w1f, eye), jnp.repeat(b1f, bs2, axis=0)],
        axis=1).astype(jnp.bfloat16)
    w2b = jnp.kron(w2.T, eye).astype(jnp.bfloat16)
    aux = jnp.broadcast_to(b2.astype(jnp.float32), (8, 128))

    outp = pl.pallas_call(
        functools.partial(_main_kernel, c=c, bs=bs2),
        out_shape=jax.ShapeDtypeStruct((hw, n), jnp.float32),
        grid=(hw // bs2,),
        in_specs=[
            pl.BlockSpec((c, bs2, n), lambda i: (0, i, 0)),
            pl.BlockSpec((k * bs2, c * bs2 + 1), lambda i: (0, 0)),
            pl.BlockSpec((bs2, k * bs2), lambda i: (0, 0)),
            pl.BlockSpec((8, 128), lambda i: (0, 0)),
        ],
        out_specs=pl.BlockSpec((bs2, n), lambda i: (i, 0)),
        compiler_params=pltpu.CompilerParams(
            dimension_semantics=("parallel",)),
    )(xp, w1a, w2b, aux)

    return outp.reshape(1, hh, ww, n).transpose(3, 0, 1, 2)


def kernel(x_nchw, w1, b1, gamma, beta, w2, b2):
    return _guide_nn_opt(x_nchw, w1, b1, gamma, beta, w2, b2)


# speedup vs baseline: 1.0989x; 1.0055x over previous
import functools

import jax
import jax.numpy as jnp
from jax import lax
from jax.experimental import pallas as pl
from jax.experimental.pallas import tpu as pltpu

_BN_EPS = 1e-5


def _pick_bs(hw, max_bs):
    bs = max_bs
    while bs > 8 and hw % bs != 0:
        bs //= 2
    return bs if hw % bs == 0 else hw


def _stats_kernel(x_ref, acc_ref, *, c, bs):
    i = pl.program_id(0)

    @pl.when(i == 0)
    def _():
        acc_ref[...] = jnp.zeros_like(acc_ref)

    xs = [x_ref[j] for j in range(c)]
    planes = xs + [xs[a] * xs[b]
                   for a in range(c) for b in range(a, c)]
    for m, v in enumerate(planes):
        r = v[0:8]
        for j in range(8, bs, 8):
            r = r + v[j:j + 8]
        acc_ref[8 * m:8 * m + 8] += r


def _main_kernel(x_ref, w1a_ref, w2b_ref, aux_ref, o_ref, *, c, bs):
    xv = x_ref[...].reshape(c * bs, x_ref.shape[2])
    ones = jnp.ones((1, xv.shape[1]), jnp.float32)
    xa = jnp.concatenate([xv, ones], axis=0).astype(jnp.bfloat16)
    h = jnp.dot(w1a_ref[...], xa,
                preferred_element_type=jnp.float32)
    r = jnp.maximum(h, 0.0)
    y = jnp.dot(w2b_ref[...], r,
                preferred_element_type=jnp.float32)
    o_ref[...] = jnp.tanh(y + aux_ref[0:1, 0:1])


@jax.jit
def _guide_nn_opt(x_nchw, w1, b1, gamma, beta, w2, b2):
    n, c, hh, ww = x_nchw.shape
    k = w1.shape[0]
    hw = hh * ww
    p = n * hw

    xp = jnp.transpose(x_nchw.astype(jnp.float32),
                       (1, 2, 3, 0)).reshape(c, hw, n)

    bs1 = _pick_bs(hw, 256)
    nm = c + c * (c + 1) // 2
    acc = pl.pallas_call(
        functools.partial(_stats_kernel, c=c, bs=bs1),
        out_shape=jax.ShapeDtypeStruct((8 * nm, n), jnp.float32),
        grid=(hw // bs1,),
        in_specs=[pl.BlockSpec((c, bs1, n), lambda i: (0, i, 0))],
        out_specs=pl.BlockSpec((8 * nm, n), lambda i: (0, 0)),
        compiler_params=pltpu.CompilerParams(
            dimension_semantics=("arbitrary",)),
    )(xp)

    gv = acc.reshape(nm, 8 * n).sum(axis=1)
    s = gv[:c]
    pairs = {}
    idx = c
    for a in range(c):
        for b in range(a, c):
            pairs[(a, b)] = pairs[(b, a)] = gv[idx]
            idx += 1
    q = jnp.stack([jnp.stack([pairs[(a, b)] for b in range(c)])
                   for a in range(c)])

    mu = s / p
    cov = q / p - mu[:, None] * mu[None, :]
    mean_y = w1 @ mu[:, None] + b1
    var_y = jnp.sum((w1 @ cov) * w1, axis=1, keepdims=True)

    scale = gamma * lax.rsqrt(var_y + _BN_EPS)
    w1f = w1 * scale
    b1f = scale * (b1 - mean_y) + beta

    bs2 = _pick_bs(hw, 64)
    eye = jnp.eye(bs2, dtype=jnp.float32)
    w1a = jnp.concatenate(
        [jnp.kron(w1f, eye), jnp.repeat(b1f, bs2, axis=0)],
        axis=1).astype(jnp.bfloat16)
    w2b = jnp.kron(w2.T, eye)
    aux = jnp.broadcast_to(b2.astype(jnp.float32), (8, 128))

    outp = pl.pallas_call(
        functools.partial(_main_kernel, c=c, bs=bs2),
        out_shape=jax.ShapeDtypeStruct((hw, n), jnp.float32),
        grid=(hw // bs2,),
        in_specs=[
            pl.BlockSpec((c, bs2, n), lambda i: (0, i, 0)),
            pl.BlockSpec((k * bs2, c * bs2 + 1), lambda i: (0, 0)),
            pl.BlockSpec((bs2, k * bs2), lambda i: (0, 0)),
            pl.BlockSpec((8, 128), lambda i: (0, 0)),
        ],
        out_specs=pl.BlockSpec((bs2, n), lambda i: (i, 0)),
        compiler_params=pltpu.CompilerParams(
            dimension_semantics=("parallel",)),
    )(xp, w1a, w2b, aux)

    return outp.reshape(1, hh, ww, n).transpose(3, 0, 1, 2)


def kernel(x_nchw, w1, b1, gamma, beta, w2, b2):
    return _guide_nn_opt(x_nchw, w1, b1, gamma, beta, w2, b2)
